# baseline (device time: 80925 ns/iter reference)
import jax
import jax.numpy as jnp
from jax import lax
from jax.experimental import pallas as pl
from jax.experimental.pallas import tpu as pltpu

N_DEV = 16
B, SQ, D = 2, 128, 512
HQ, DH = 8, 64
ROWS = B * SQ
CHUNK = ROWS // N_DEV


def kernel(x, Wq, Wo, Wk, Wv):
    x2 = x.reshape(ROWS, D)

    def body(x_ref, wq_ref, wk_ref, wv_ref, wo_ref, out_ref,
             attn_ref, part_ref, comm_ref,
             rs_send, rs_recv, ag_send, ag_recv):
        me = lax.axis_index("i")
        right = lax.rem(me + 1, N_DEV)
        left = lax.rem(me + N_DEV - 1, N_DEV)

        xv = x_ref[:, :]
        q = jnp.dot(xv, wq_ref[:, :], preferred_element_type=jnp.float32)
        k = jnp.dot(xv, wk_ref[:, :], preferred_element_type=jnp.float32)
        v = jnp.dot(xv, wv_ref[:, :], preferred_element_type=jnp.float32)

        for b in range(B):
            r0 = b * SQ
            for h in range(HQ):
                c0 = h * DH
                qh = q[r0:r0 + SQ, c0:c0 + DH]
                kh = k[r0:r0 + SQ, c0:c0 + DH]
                vh = v[r0:r0 + SQ, c0:c0 + DH]
                s = lax.dot_general(
                    qh, kh, (((1,), (1,)), ((), ())),
                    preferred_element_type=jnp.float32) * 0.125
                m = jnp.max(s, axis=1, keepdims=True)
                p = jnp.exp(s - m)
                l = jnp.sum(p, axis=1, keepdims=True)
                o = jnp.dot(p, vh, preferred_element_type=jnp.float32) / l
                attn_ref[r0:r0 + SQ, c0:c0 + DH] = o
            part_ref[r0:r0 + SQ, :] = jnp.dot(
                attn_ref[r0:r0 + SQ, :], wo_ref[:, :],
                preferred_element_type=jnp.float32)

        bar = pltpu.get_barrier_semaphore()
        for nbr in (left, right):
            pl.semaphore_signal(bar, inc=1, device_id=(nbr,),
                                device_id_type=pl.DeviceIdType.MESH)
        pl.semaphore_wait(bar, 2)

        comm_ref[0, :, :] = part_ref[pl.ds(me * CHUNK, CHUNK), :]
        for hop in range(N_DEV - 1):
            rdma = pltpu.make_async_remote_copy(
                src_ref=comm_ref.at[hop],
                dst_ref=comm_ref.at[hop + 1],
                send_sem=rs_send.at[hop],
                recv_sem=rs_recv.at[hop],
                device_id=(right,),
                device_id_type=pl.DeviceIdType.MESH,
            )
            rdma.start()
            rdma.wait()
            c = lax.rem(me - hop - 1 + 2 * N_DEV, N_DEV)
            comm_ref[hop + 1, :, :] = (
                comm_ref[hop + 1, :, :]
                + part_ref[pl.ds(c * CHUNK, CHUNK), :])

        own = lax.rem(me + 1, N_DEV)
        out_ref[pl.ds(own * CHUNK, CHUNK), :] = comm_ref[N_DEV - 1, :, :]
        for hop in range(N_DEV - 1):
            c = lax.rem(me + 1 - hop + 2 * N_DEV, N_DEV)
            rdma = pltpu.make_async_remote_copy(
                src_ref=out_ref.at[pl.ds(c * CHUNK, CHUNK), :],
                dst_ref=out_ref.at[pl.ds(c * CHUNK, CHUNK), :],
                send_sem=ag_send.at[hop],
                recv_sem=ag_recv.at[hop],
                device_id=(right,),
                device_id_type=pl.DeviceIdType.MESH,
            )
            rdma.start()
            rdma.wait()

    out = pl.pallas_call(
        body,
        out_shape=jax.ShapeDtypeStruct((ROWS, D), jnp.float32),
        in_specs=[pl.BlockSpec(memory_space=pltpu.VMEM)] * 5,
        out_specs=pl.BlockSpec(memory_space=pltpu.VMEM),
        scratch_shapes=[
            pltpu.VMEM((ROWS, D), jnp.float32),
            pltpu.VMEM((ROWS, D), jnp.float32),
            pltpu.VMEM((N_DEV, CHUNK, D), jnp.float32),
            pltpu.SemaphoreType.DMA((N_DEV,)),
            pltpu.SemaphoreType.DMA((N_DEV,)),
            pltpu.SemaphoreType.DMA((N_DEV,)),
            pltpu.SemaphoreType.DMA((N_DEV,)),
        ],
        compiler_params=pltpu.CompilerParams(collective_id=0),
    )(x2, Wq, Wk, Wv, Wo)
    return out.reshape(B, SQ, D)


# device time: 31334 ns/iter; 2.5827x vs baseline; 2.5827x over previous
import jax
import jax.numpy as jnp
from jax import lax
from jax.experimental import pallas as pl
from jax.experimental.pallas import tpu as pltpu

N_DEV = 16
B, SQ, D = 2, 128, 512
HQ, DH = 8, 64
ROWS = B * SQ
CHUNK = ROWS // N_DEV


def kernel(x, Wq, Wo, Wk, Wv):
    x2 = x.reshape(ROWS, D)

    def body(x_ref, wq_ref, wk_ref, wv_ref, wo_ref, out_ref,
             attn_ref, part_ref, rsbuf_ref,
             rs_send, rs_recv, ag_send, ag_recv):
        me = lax.axis_index("i")

        xv = x_ref[:, :]
        q = jnp.dot(xv, wq_ref[:, :], preferred_element_type=jnp.float32)
        k = jnp.dot(xv, wk_ref[:, :], preferred_element_type=jnp.float32)
        v = jnp.dot(xv, wv_ref[:, :], preferred_element_type=jnp.float32)

        for b in range(B):
            r0 = b * SQ
            for h in range(HQ):
                c0 = h * DH
                qh = q[r0:r0 + SQ, c0:c0 + DH]
                kh = k[r0:r0 + SQ, c0:c0 + DH]
                vh = v[r0:r0 + SQ, c0:c0 + DH]
                s = lax.dot_general(
                    qh, kh, (((1,), (1,)), ((), ())),
                    preferred_element_type=jnp.float32) * 0.125
                m = jnp.max(s, axis=1, keepdims=True)
                p = jnp.exp(s - m)
                l = jnp.sum(p, axis=1, keepdims=True)
                o = jnp.dot(p, vh, preferred_element_type=jnp.float32) / l
                attn_ref[r0:r0 + SQ, c0:c0 + DH] = o
            part_ref[r0:r0 + SQ, :] = jnp.dot(
                attn_ref[r0:r0 + SQ, :], wo_ref[:, :],
                preferred_element_type=jnp.float32)

        bar = pltpu.get_barrier_semaphore()
        for d in range(1, N_DEV):
            t = lax.rem(me + d, N_DEV)
            pl.semaphore_signal(bar, inc=1, device_id=(t,),
                                device_id_type=pl.DeviceIdType.MESH)
        pl.semaphore_wait(bar, N_DEV - 1)

        rsbuf_ref[pl.ds(me, 1), :, :] = part_ref[
            pl.ds(me * CHUNK, CHUNK), :].reshape(1, CHUNK, D)
        rs_sends = []
        for d in range(1, N_DEV):
            t = lax.rem(me + d, N_DEV)
            rdma = pltpu.make_async_remote_copy(
                src_ref=part_ref.at[pl.ds(t * CHUNK, CHUNK), :],
                dst_ref=rsbuf_ref.at[me],
                send_sem=rs_send.at[d],
                recv_sem=rs_recv.at[me],
                device_id=(t,),
                device_id_type=pl.DeviceIdType.MESH,
            )
            rdma.start()
            rs_sends.append(rdma)
        for d in range(1, N_DEV):
            s_ = lax.rem(me + d, N_DEV)
            recv = pltpu.make_async_remote_copy(
                src_ref=part_ref.at[pl.ds(s_ * CHUNK, CHUNK), :],
                dst_ref=rsbuf_ref.at[s_],
                send_sem=rs_send.at[d],
                recv_sem=rs_recv.at[s_],
                device_id=(s_,),
                device_id_type=pl.DeviceIdType.MESH,
            )
            recv.wait_recv()

        acc = rsbuf_ref[0, :, :]
        for s_ in range(1, N_DEV):
            acc = acc + rsbuf_ref[s_, :, :]

        my_rows = pl.ds(me * CHUNK, CHUNK)
        out_ref[my_rows, :] = acc
        ag_sends = []
        for d in range(1, N_DEV):
            t = lax.rem(me + d, N_DEV)
            rdma = pltpu.make_async_remote_copy(
                src_ref=out_ref.at[my_rows, :],
                dst_ref=out_ref.at[my_rows, :],
                send_sem=ag_send.at[d],
                recv_sem=ag_recv.at[me],
                device_id=(t,),
                device_id_type=pl.DeviceIdType.MESH,
            )
            rdma.start()
            ag_sends.append(rdma)
        for d in range(1, N_DEV):
            s_ = lax.rem(me + d, N_DEV)
            recv = pltpu.make_async_remote_copy(
                src_ref=out_ref.at[pl.ds(s_ * CHUNK, CHUNK), :],
                dst_ref=out_ref.at[pl.ds(s_ * CHUNK, CHUNK), :],
                send_sem=ag_send.at[d],
                recv_sem=ag_recv.at[s_],
                device_id=(s_,),
                device_id_type=pl.DeviceIdType.MESH,
            )
            recv.wait_recv()

        for rdma in rs_sends:
            rdma.wait_send()
        for rdma in ag_sends:
            rdma.wait_send()

    out = pl.pallas_call(
        body,
        out_shape=jax.ShapeDtypeStruct((ROWS, D), jnp.float32),
        in_specs=[pl.BlockSpec(memory_space=pltpu.VMEM)] * 5,
        out_specs=pl.BlockSpec(memory_space=pltpu.VMEM),
        scratch_shapes=[
            pltpu.VMEM((ROWS, D), jnp.float32),
            pltpu.VMEM((ROWS, D), jnp.float32),
            pltpu.VMEM((N_DEV, CHUNK, D), jnp.float32),
            pltpu.SemaphoreType.DMA((N_DEV,)),
            pltpu.SemaphoreType.DMA((N_DEV,)),
            pltpu.SemaphoreType.DMA((N_DEV,)),
            pltpu.SemaphoreType.DMA((N_DEV,)),
        ],
        compiler_params=pltpu.CompilerParams(collective_id=0),
    )(x2, Wq, Wk, Wv, Wo)
    return out.reshape(B, SQ, D)


# device time: 12077 ns/iter; 6.7008x vs baseline; 2.5945x over previous
import jax
import jax.numpy as jnp
from jax import lax
from jax.experimental import pallas as pl
from jax.experimental.pallas import tpu as pltpu

N_DEV = 16
B, SQ, D = 2, 128, 512
HQ, DH = 8, 64
ROWS = B * SQ
CHUNK = ROWS // N_DEV
CPB = SQ // CHUNK


def kernel(x, Wq, Wo, Wk, Wv):
    x2 = x.reshape(ROWS, D)

    def body(x_ref, wq_ref, wk_ref, wv_ref, wo_ref, out_ref,
             attn_ref, part_ref, rsbuf_ref,
             rs_send, rs_recv, ag_send, ag_recv):
        me = lax.axis_index("i")

        bar = pltpu.get_barrier_semaphore()
        for d in range(1, N_DEV):
            t = lax.rem(me + d, N_DEV)
            pl.semaphore_signal(bar, inc=1, device_id=(t,),
                                device_id_type=pl.DeviceIdType.MESH)
        pl.semaphore_wait(bar, N_DEV - 1)

        def rs_descriptor(d):
            t = lax.rem(me + d, N_DEV)
            return t, pltpu.make_async_remote_copy(
                src_ref=part_ref.at[pl.ds(t * CHUNK, CHUNK), :],
                dst_ref=rsbuf_ref.at[me],
                send_sem=rs_send.at[d],
                recv_sem=rs_recv.at[me],
                device_id=(t,),
                device_id_type=pl.DeviceIdType.MESH,
            )

        xv = x_ref[:, :]
        q = jnp.dot(xv, wq_ref[:, :], preferred_element_type=jnp.float32)
        k = jnp.dot(xv, wk_ref[:, :], preferred_element_type=jnp.float32)
        v = jnp.dot(xv, wv_ref[:, :], preferred_element_type=jnp.float32)

        for b in range(B):
            r0 = b * SQ
            for h in range(HQ):
                c0 = h * DH
                qh = q[r0:r0 + SQ, c0:c0 + DH]
                kh = k[r0:r0 + SQ, c0:c0 + DH]
                vh = v[r0:r0 + SQ, c0:c0 + DH]
                s = lax.dot_general(
                    qh, kh, (((1,), (1,)), ((), ())),
                    preferred_element_type=jnp.float32) * 0.125
                m = jnp.max(s, axis=1, keepdims=True)
                p = jnp.exp(s - m)
                l = jnp.sum(p, axis=1, keepdims=True)
                o = jnp.dot(p, vh, preferred_element_type=jnp.float32) / l
                attn_ref[r0:r0 + SQ, c0:c0 + DH] = o
            part_ref[r0:r0 + SQ, :] = jnp.dot(
                attn_ref[r0:r0 + SQ, :], wo_ref[:, :],
                preferred_element_type=jnp.float32)
            if b == 0:
                for d in range(1, N_DEV):
                    t, rdma = rs_descriptor(d)
                    pl.when(t < CPB)(rdma.start)

        for d in range(1, N_DEV):
            t, rdma = rs_descriptor(d)
            pl.when(t >= CPB)(rdma.start)
        rsbuf_ref[pl.ds(me, 1), :, :] = part_ref[
            pl.ds(me * CHUNK, CHUNK), :].reshape(1, CHUNK, D)

        acc = rsbuf_ref[pl.ds(me, 1), :, :].reshape(CHUNK, D)
        for d in range(1, N_DEV):
            s_ = lax.rem(me + d, N_DEV)
            recv = pltpu.make_async_remote_copy(
                src_ref=part_ref.at[pl.ds(s_ * CHUNK, CHUNK), :],
                dst_ref=rsbuf_ref.at[s_],
                send_sem=rs_send.at[d],
                recv_sem=rs_recv.at[s_],
                device_id=(s_,),
                device_id_type=pl.DeviceIdType.MESH,
            )
            recv.wait_recv()
            acc = acc + rsbuf_ref[pl.ds(s_, 1), :, :].reshape(CHUNK, D)

        my_rows = pl.ds(me * CHUNK, CHUNK)
        out_ref[my_rows, :] = acc
        ag_sends = []
        for d in range(1, N_DEV):
            t = lax.rem(me + d, N_DEV)
            rdma = pltpu.make_async_remote_copy(
                src_ref=out_ref.at[my_rows, :],
                dst_ref=out_ref.at[my_rows, :],
                send_sem=ag_send.at[d],
                recv_sem=ag_recv.at[me],
                device_id=(t,),
                device_id_type=pl.DeviceIdType.MESH,
            )
            rdma.start()
            ag_sends.append(rdma)
        for d in range(1, N_DEV):
            s_ = lax.rem(me + d, N_DEV)
            recv = pltpu.make_async_remote_copy(
                src_ref=out_ref.at[pl.ds(s_ * CHUNK, CHUNK), :],
                dst_ref=out_ref.at[pl.ds(s_ * CHUNK, CHUNK), :],
                send_sem=ag_send.at[d],
                recv_sem=ag_recv.at[s_],
                device_id=(s_,),
                device_id_type=pl.DeviceIdType.MESH,
            )
            recv.wait_recv()

        for d in range(1, N_DEV):
            _, rdma = rs_descriptor(d)
            rdma.wait_send()
        for rdma in ag_sends:
            rdma.wait_send()

    out = pl.pallas_call(
        body,
        out_shape=jax.ShapeDtypeStruct((ROWS, D), jnp.float32),
        in_specs=[pl.BlockSpec(memory_space=pltpu.VMEM)] * 5,
        out_specs=pl.BlockSpec(memory_space=pltpu.VMEM),
        scratch_shapes=[
            pltpu.VMEM((ROWS, D), jnp.float32),
            pltpu.VMEM((ROWS, D), jnp.float32),
            pltpu.VMEM((N_DEV, CHUNK, D), jnp.float32),
            pltpu.SemaphoreType.DMA((N_DEV,)),
            pltpu.SemaphoreType.DMA((N_DEV,)),
            pltpu.SemaphoreType.DMA((N_DEV,)),
            pltpu.SemaphoreType.DMA((N_DEV,)),
        ],
        compiler_params=pltpu.CompilerParams(collective_id=0),
    )(x2, Wq, Wk, Wv, Wo)
    return out.reshape(B, SQ, D)
